# initial kernel scaffold (unmeasured)
import jax
import jax.numpy as jnp
from jax import lax
from jax.experimental import pallas as pl
from jax.experimental.pallas import tpu as pltpu

N_DEV = 32
HQ = 8
DH = 128
SQ = 512
D = 1024
SKV = 2048
CHUNK = SQ // N_DEV
SCALE = 0.08838834764831843


def kernel(x, Wq, Wo, K_ext, V_ext):
    my_pos = lax.axis_index("i")
    h0 = my_pos * HQ
    K_sl = lax.dynamic_slice(K_ext, (0, 0, h0, 0), (1, SKV, HQ, DH)).reshape(SKV, HQ, DH)
    V_sl = lax.dynamic_slice(V_ext, (0, 0, h0, 0), (1, SKV, HQ, DH)).reshape(SKV, HQ, DH)

    def body(x_ref, wq_ref, wo_ref, k_ref, v_ref, out_ref,
             acc_ref, rs_ref, rs_send_sems, rs_recv_sems,
             ag_send_sems, ag_recv_sems):
        my = lax.axis_index("i")
        left = (my + N_DEV - 1) % N_DEV
        right = (my + 1) % N_DEV

        barrier = pltpu.get_barrier_semaphore()
        pl.semaphore_signal(barrier, inc=1, device_id=(left,),
                            device_id_type=pl.DeviceIdType.MESH)
        pl.semaphore_signal(barrier, inc=1, device_id=(right,),
                            device_id_type=pl.DeviceIdType.MESH)
        pl.semaphore_wait(barrier, 2)

        xq = x_ref[0]
        q = jnp.dot(xq, wq_ref[...],
                    preferred_element_type=jnp.float32)
        outs = []
        for h in range(HQ):
            qh = q[:, h * DH:(h + 1) * DH]
            kh = k_ref[:, h, :]
            vh = v_ref[:, h, :]
            s = lax.dot_general(qh, kh, (((1,), (1,)), ((), ())),
                                preferred_element_type=jnp.float32) * SCALE
            m = jnp.max(s, axis=1, keepdims=True)
            p = jnp.exp(s - m)
            l = jnp.sum(p, axis=1, keepdims=True)
            o = jnp.dot(p, vh, preferred_element_type=jnp.float32) / l
            outs.append(o)
        attn = jnp.concatenate(outs, axis=1)
        acc_ref[...] = jnp.dot(attn, wo_ref[...],
                               preferred_element_type=jnp.float32)

        for t in range(N_DEV - 1):
            send_idx = (my - t + N_DEV) % N_DEV
            rdma = pltpu.make_async_remote_copy(
                src_ref=acc_ref.at[pl.ds(send_idx * CHUNK, CHUNK), :],
                dst_ref=rs_ref.at[t],
                send_sem=rs_send_sems.at[t],
                recv_sem=rs_recv_sems.at[t],
                device_id=(right,),
                device_id_type=pl.DeviceIdType.MESH,
            )
            rdma.start()
            rdma.wait()
            recv_idx = (my - t - 1 + N_DEV) % N_DEV
            sl = pl.ds(recv_idx * CHUNK, CHUNK)
            acc_ref[sl, :] = acc_ref[sl, :] + rs_ref[t]

        own = (my + 1) % N_DEV
        own_sl = pl.ds(own * CHUNK, CHUNK)
        out_ref[0, own_sl, :] = acc_ref[own_sl, :]

        for t in range(N_DEV - 1):
            send_idx = (my + 1 - t + 2 * N_DEV) % N_DEV
            recv_idx = (my - t + 2 * N_DEV) % N_DEV
            rdma = pltpu.make_async_remote_copy(
                src_ref=out_ref.at[0, pl.ds(send_idx * CHUNK, CHUNK), :],
                dst_ref=out_ref.at[0, pl.ds(recv_idx * CHUNK, CHUNK), :],
                send_sem=ag_send_sems.at[t],
                recv_sem=ag_recv_sems.at[t],
                device_id=(right,),
                device_id_type=pl.DeviceIdType.MESH,
            )
            rdma.start()
            rdma.wait()

    return pl.pallas_call(
        body,
        out_shape=jax.ShapeDtypeStruct((1, SQ, D), jnp.float32),
        in_specs=[pl.BlockSpec(memory_space=pltpu.VMEM)] * 5,
        out_specs=pl.BlockSpec(memory_space=pltpu.VMEM),
        scratch_shapes=[
            pltpu.VMEM((SQ, D), jnp.float32),
            pltpu.VMEM((N_DEV - 1, CHUNK, D), jnp.float32),
            pltpu.SemaphoreType.DMA((N_DEV - 1,)),
            pltpu.SemaphoreType.DMA((N_DEV - 1,)),
            pltpu.SemaphoreType.DMA((N_DEV - 1,)),
            pltpu.SemaphoreType.DMA((N_DEV - 1,)),
        ],
        compiler_params=pltpu.CompilerParams(collective_id=0),
    )(x, Wq, Wo, K_sl, V_sl)


# baseline (device time: 232766 ns/iter reference)
import os

import jax
import jax.numpy as jnp
from jax import lax
from jax.experimental import pallas as pl
from jax.experimental.pallas import tpu as pltpu

_SKIP_RING = os.environ.get("KDBG_SKIP_RING") == "1"
_SKIP_ATTN = os.environ.get("KDBG_SKIP_ATTN") == "1"

N_DEV = 32
HQ = 8
DH = 128
SQ = 512
D = 1024
SKV = 2048
CHUNK = SQ // N_DEV
NSEM = 8
SCALE = 0.08838834764831843


def kernel(x, Wq, Wo, K_ext, V_ext):
    my_pos = lax.axis_index("i")
    h0 = my_pos * HQ
    K_sl = jnp.transpose(
        lax.dynamic_slice(K_ext, (0, 0, h0, 0), (1, SKV, HQ, DH)).reshape(SKV, HQ, DH),
        (1, 0, 2))
    V_sl = jnp.transpose(
        lax.dynamic_slice(V_ext, (0, 0, h0, 0), (1, SKV, HQ, DH)).reshape(SKV, HQ, DH),
        (1, 0, 2))

    def body(x_ref, wq_ref, wo_ref, k_ref, v_ref, out_ref,
             q_ref, acc_ref, rs_ref, rs_send, rs_recv, ag_send, ag_recv):
        my = lax.axis_index("i")

        q_ref[...] = jnp.dot(x_ref[0], wq_ref[...],
                             preferred_element_type=jnp.float32)
        acc_ref[...] = jnp.zeros((SQ, D), jnp.float32)

        def head_body(h, carry):
            hsl = pl.ds(h * DH, DH)
            qh = q_ref[:, hsl]
            kh = k_ref[h]
            vh = v_ref[h]
            s = lax.dot_general(qh, kh, (((1,), (1,)), ((), ())),
                                preferred_element_type=jnp.float32) * SCALE
            m = jnp.max(s, axis=1, keepdims=True)
            p = jnp.exp(s - m)
            l = jnp.sum(p, axis=1, keepdims=True)
            o = jnp.dot(p, vh, preferred_element_type=jnp.float32) / l
            acc_ref[...] += jnp.dot(o, wo_ref[hsl, :],
                                    preferred_element_type=jnp.float32)
            return carry

        if not _SKIP_ATTN:
            lax.fori_loop(0, HQ, head_body, 0)

        if _SKIP_RING:
            out_ref[0, :, :] = acc_ref[...]
            return

        left = (my + N_DEV - 1) % N_DEV
        right = (my + 1) % N_DEV

        for t in range(N_DEV - 1):
            send_idx = (my - t + N_DEV) % N_DEV
            rdma = pltpu.make_async_remote_copy(
                src_ref=acc_ref.at[pl.ds(send_idx * CHUNK, CHUNK), :],
                dst_ref=rs_ref.at[t % NSEM],
                send_sem=rs_send.at[t % NSEM],
                recv_sem=rs_recv.at[t % NSEM],
                device_id=(right,),
                device_id_type=pl.DeviceIdType.MESH,
            )
            rdma.start()
            rdma.wait()
            recv_idx = (my - t - 1 + N_DEV) % N_DEV
            sl = pl.ds(recv_idx * CHUNK, CHUNK)
            acc_ref[sl, :] = acc_ref[sl, :] + rs_ref[t % NSEM]

        own = (my + 1) % N_DEV
        own_sl = pl.ds(own * CHUNK, CHUNK)
        out_ref[0, own_sl, :] = acc_ref[own_sl, :]

        for t in range(N_DEV - 1):
            send_idx = (my + 1 - t + 2 * N_DEV) % N_DEV
            csl = pl.ds(send_idx * CHUNK, CHUNK)
            rdma = pltpu.make_async_remote_copy(
                src_ref=out_ref.at[0, csl, :],
                dst_ref=out_ref.at[0, csl, :],
                send_sem=ag_send.at[t % NSEM],
                recv_sem=ag_recv.at[t % NSEM],
                device_id=(right,),
                device_id_type=pl.DeviceIdType.MESH,
            )
            rdma.start()
            rdma.wait()

    return pl.pallas_call(
        body,
        out_shape=jax.ShapeDtypeStruct((1, SQ, D), jnp.float32),
        in_specs=[pl.BlockSpec(memory_space=pltpu.VMEM)] * 5,
        out_specs=pl.BlockSpec(memory_space=pltpu.VMEM),
        scratch_shapes=[
            pltpu.VMEM((SQ, D), jnp.float32),
            pltpu.VMEM((SQ, D), jnp.float32),
            pltpu.VMEM((NSEM, CHUNK, D), jnp.float32),
            pltpu.SemaphoreType.DMA((NSEM,)),
            pltpu.SemaphoreType.DMA((NSEM,)),
            pltpu.SemaphoreType.DMA((NSEM,)),
            pltpu.SemaphoreType.DMA((NSEM,)),
        ],
        compiler_params=pltpu.CompilerParams(
            vmem_limit_bytes=100 * 1024 * 1024,
        ),
    )(x, Wq, Wo, K_sl, V_sl)
